# Initial kernel scaffold; baseline (speedup 1.0000x reference)
import jax
import jax.numpy as jnp
from jax.experimental import pallas as pl


def kernel(x_nchw, conv_w9, conv_scale, conv_shift, valid_mask, fc_w, fc_b):
    raise NotImplementedError("write your pallas kernel here")


# speedup vs baseline: 1.4461x; 1.4461x over previous
import jax
import jax.numpy as jnp
from jax.experimental import pallas as pl
from jax.experimental.pallas import tpu as pltpu


OUTNUM = 14
GROUP = 4
C_IN = 2048
C_MID = 256
FC_IN = 1024
FC_PAD = 128

IMG_H = 8
IMG_W = 8
IMG = IMG_H * IMG_W
TB = 8
M_ROWS = TB * IMG
FRONT = 16
BACK = 16
RB = FRONT + M_ROWS + BACK
FC_ROWS = TB // GROUP


def _fused_kernel(x_ref, w_ref, scale_ref, shift_ref, mask_ref, fcw_ref,
                  fcb_ref, o_ref, acc_ref):
    acc_ref[...] = jnp.zeros_like(acc_ref)
    for di in range(3):
        for dj in range(3):
            off = (di - 1) * IMG_W + (dj - 1)
            xs = x_ref[FRONT + off:FRONT + off + M_ROWS, :]
            acc_ref[...] += jnp.dot(xs, w_ref[di * 3 + dj],
                                    preferred_element_type=jnp.float32)
    y = jnp.maximum(acc_ref[...] * scale_ref[...] + shift_ref[...], 0.0)
    y = y * mask_ref[...]
    pooled = [jnp.max(y[m * IMG:(m + 1) * IMG, :], axis=0, keepdims=True)
              for m in range(TB)]
    rows = [jnp.concatenate(pooled[g * GROUP:(g + 1) * GROUP], axis=1)
            for g in range(FC_ROWS)]
    feats = jnp.concatenate(rows, axis=0).astype(jnp.bfloat16)
    o_ref[0] = (jnp.dot(feats, fcw_ref[...],
                        preferred_element_type=jnp.float32) + fcb_ref[...])


def kernel(x_nchw, conv_w9, conv_scale, conv_shift, valid_mask, fc_w, fc_b):
    N, C, H, W = x_nchw.shape
    assert C == C_IN and H == 7 and W == 7 and N % TB == 0

    x = jnp.transpose(x_nchw, (0, 2, 3, 1)).astype(jnp.bfloat16)
    x = jnp.pad(x, ((0, 0), (0, 1), (1, 0), (0, 0)))
    x = x.reshape(N, IMG, C_IN)
    nblk = N // TB
    x = x.reshape(nblk, TB * IMG, C_IN)
    x = jnp.pad(x, ((0, 0), (FRONT, BACK), (0, 0)))
    x = x.reshape(nblk * RB, C_IN)

    G = N // GROUP
    out = pl.pallas_call(
        _fused_kernel,
        out_shape=jax.ShapeDtypeStruct((nblk, FC_ROWS, FC_PAD), jnp.float32),
        grid=(nblk,),
        in_specs=[
            pl.BlockSpec((RB, C_IN), lambda i: (i, 0)),
            pl.BlockSpec((9, C_IN, C_MID), lambda i: (0, 0, 0)),
            pl.BlockSpec((1, C_MID), lambda i: (0, 0)),
            pl.BlockSpec((1, C_MID), lambda i: (0, 0)),
            pl.BlockSpec((M_ROWS, 1), lambda i: (0, 0)),
            pl.BlockSpec((FC_IN, FC_PAD), lambda i: (0, 0)),
            pl.BlockSpec((1, FC_PAD), lambda i: (0, 0)),
        ],
        out_specs=pl.BlockSpec((1, FC_ROWS, FC_PAD), lambda i: (i, 0, 0)),
        scratch_shapes=[pltpu.VMEM((M_ROWS, C_MID), jnp.float32)],
        compiler_params=pltpu.CompilerParams(
            dimension_semantics=("parallel",),
            vmem_limit_bytes=64 * 1024 * 1024),
    )(x, conv_w9, conv_scale, conv_shift, valid_mask, fc_w, fc_b)

    return out.reshape(G, FC_PAD)[:, :OUTNUM]
